# baseline (speedup 1.0000x reference)
import numpy as np

import jax
import jax.numpy as jnp
from jax.experimental import pallas as pl
from jax.experimental.pallas import tpu as pltpu

N_NODES = 16384
OBS_DIM = 4096
NUM_EXPERTS = 10
TOP_K = 3
ROW_BLOCK = 512

_EPS = np.asarray(jax.random.normal(jax.random.key(42), (1, NUM_EXPERTS),
                                    dtype=jnp.float32))


NBUF = 6
N_CHUNKS = N_NODES // ROW_BLOCK


def _hive_kernel(x_hbm, wg_ref, wn_ref, bg_ref, bn_ref, eps_ref,
                 w_out, l_out, v_out, i_out, bufs, sems, acc_ref):
    def _start(g):
        i = g % NBUF
        pltpu.make_async_copy(
            x_hbm.at[pl.ds(g * ROW_BLOCK, ROW_BLOCK)],
            bufs.at[i], sems.at[i]).start()

    for g in range(NBUF):
        _start(g)

    acc_ref[...] = jnp.zeros_like(acc_ref)
    for g in range(N_CHUNKS):
        i = g % NBUF
        pltpu.make_async_copy(
            x_hbm.at[pl.ds(g * ROW_BLOCK, ROW_BLOCK)],
            bufs.at[i], sems.at[i]).wait()
        acc_ref[...] += jnp.sum(bufs[i], axis=0, keepdims=True)
        if g + NBUF < N_CHUNKS:
            _start(g + NBUF)

    if True:
        gs = acc_ref[...] * (1.0 / N_NODES)
        dn = (((1,), (1,)), ((), ()))
        clean = jax.lax.dot_general(
            gs, wg_ref[...], dn,
            preferred_element_type=jnp.float32) + bg_ref[...]
        raw = jax.lax.dot_general(
            gs, wn_ref[...], dn,
            preferred_element_type=jnp.float32) + bn_ref[...]
        noise_std = jnp.logaddexp(raw, 0.0)
        logits = clean + eps_ref[...] * noise_std
        m = jnp.max(logits, axis=-1, keepdims=True)
        e = jnp.exp(logits - m)
        weights = e / jnp.sum(e, axis=-1, keepdims=True)
        l_out[...] = logits
        w_out[...] = weights

        idxs = jax.lax.broadcasted_iota(jnp.int32, (1, NUM_EXPERTS), 1)
        cur = weights
        vals, inds = [], []
        for _ in range(TOP_K):
            v = jnp.max(cur, axis=-1, keepdims=True)
            a = jnp.min(jnp.where(cur == v, idxs, NUM_EXPERTS),
                        axis=-1, keepdims=True)
            cur = jnp.where(idxs == a, -jnp.inf, cur)
            vals.append(v)
            inds.append(a)
        v_out[...] = jnp.concatenate(vals, axis=-1)
        i_out[...] = jnp.concatenate(inds, axis=-1)


@jax.jit
def _run(ip_x, w_gating, b_gating, w_noise, b_noise):
    vm = lambda: pl.BlockSpec(memory_space=pltpu.MemorySpace.VMEM)
    out = pl.pallas_call(
        _hive_kernel,
        in_specs=[
            pl.BlockSpec(memory_space=pltpu.MemorySpace.HBM),
            vm(), vm(), vm(), vm(), vm(),
        ],
        out_specs=[vm(), vm(), vm(), vm()],
        out_shape=[
            jax.ShapeDtypeStruct((1, NUM_EXPERTS), jnp.float32),
            jax.ShapeDtypeStruct((1, NUM_EXPERTS), jnp.float32),
            jax.ShapeDtypeStruct((1, TOP_K), jnp.float32),
            jax.ShapeDtypeStruct((1, TOP_K), jnp.int32),
        ],
        scratch_shapes=[
            pltpu.VMEM((NBUF, ROW_BLOCK, OBS_DIM), jnp.float32),
            pltpu.SemaphoreType.DMA((NBUF,)),
            pltpu.VMEM((1, OBS_DIM), jnp.float32),
        ],
    )(ip_x, w_gating.T, w_noise.T,
      b_gating.reshape(1, NUM_EXPERTS), b_noise.reshape(1, NUM_EXPERTS),
      jnp.asarray(_EPS))
    weights, logits, top_k_vals, top_k_indices = out
    return weights, logits, top_k_vals, top_k_indices


def kernel(ip_x, w_gating, b_gating, w_noise, b_noise, top_k):
    del top_k
    return _run(ip_x, w_gating, b_gating, w_noise, b_noise)


# speedup vs baseline: 1.0209x; 1.0209x over previous
import numpy as np

import jax
import jax.numpy as jnp
from jax.experimental import pallas as pl
from jax.experimental.pallas import tpu as pltpu

N_NODES = 16384
OBS_DIM = 4096
NUM_EXPERTS = 10
TOP_K = 3
ROW_BLOCK = 512

_EPS = np.asarray(jax.random.normal(jax.random.key(42), (1, NUM_EXPERTS),
                                    dtype=jnp.float32))


def _hive_kernel(x_ref, wg_ref, wn_ref, bg_ref, bn_ref, eps_ref,
                 w_out, l_out, v_out, i_out, acc_ref):
    step = pl.program_id(0)

    @pl.when(step == 0)
    def _init():
        acc_ref[...] = jnp.zeros_like(acc_ref)

    acc_ref[...] += jnp.sum(x_ref[...], axis=0, keepdims=True)

    @pl.when(step == pl.num_programs(0) - 1)
    def _finish():
        gs = acc_ref[...] * (1.0 / N_NODES)
        dn = (((1,), (1,)), ((), ()))
        clean = jax.lax.dot_general(
            gs, wg_ref[...], dn,
            preferred_element_type=jnp.float32) + bg_ref[...]
        raw = jax.lax.dot_general(
            gs, wn_ref[...], dn,
            preferred_element_type=jnp.float32) + bn_ref[...]
        noise_std = jnp.logaddexp(raw, 0.0)
        logits = clean + eps_ref[...] * noise_std
        m = jnp.max(logits, axis=-1, keepdims=True)
        e = jnp.exp(logits - m)
        weights = e / jnp.sum(e, axis=-1, keepdims=True)
        l_out[...] = logits
        w_out[...] = weights

        idxs = jax.lax.broadcasted_iota(jnp.int32, (1, NUM_EXPERTS), 1)
        cur = weights
        vals, inds = [], []
        for _ in range(TOP_K):
            v = jnp.max(cur, axis=-1, keepdims=True)
            a = jnp.min(jnp.where(cur == v, idxs, NUM_EXPERTS),
                        axis=-1, keepdims=True)
            cur = jnp.where(idxs == a, -jnp.inf, cur)
            vals.append(v)
            inds.append(a)
        v_out[...] = jnp.concatenate(vals, axis=-1)
        i_out[...] = jnp.concatenate(inds, axis=-1)


@jax.jit
def _run(ip_x, w_gating, b_gating, w_noise, b_noise):
    n_steps = N_NODES // ROW_BLOCK
    full = lambda shape: pl.BlockSpec(shape, lambda i: (0,) * len(shape))
    out = pl.pallas_call(
        _hive_kernel,
        grid=(n_steps,),
        in_specs=[
            pl.BlockSpec((ROW_BLOCK, OBS_DIM), lambda i: (i, 0)),
            full((NUM_EXPERTS, OBS_DIM)),
            full((NUM_EXPERTS, OBS_DIM)),
            full((1, NUM_EXPERTS)),
            full((1, NUM_EXPERTS)),
            full((1, NUM_EXPERTS)),
        ],
        out_specs=[
            full((1, NUM_EXPERTS)),
            full((1, NUM_EXPERTS)),
            full((1, TOP_K)),
            full((1, TOP_K)),
        ],
        out_shape=[
            jax.ShapeDtypeStruct((1, NUM_EXPERTS), jnp.float32),
            jax.ShapeDtypeStruct((1, NUM_EXPERTS), jnp.float32),
            jax.ShapeDtypeStruct((1, TOP_K), jnp.float32),
            jax.ShapeDtypeStruct((1, TOP_K), jnp.int32),
        ],
        scratch_shapes=[pltpu.VMEM((1, OBS_DIM), jnp.float32)],
        compiler_params=pltpu.CompilerParams(
            dimension_semantics=("arbitrary",),
        ),
    )(ip_x, w_gating.T, w_noise.T,
      b_gating.reshape(1, NUM_EXPERTS), b_noise.reshape(1, NUM_EXPERTS),
      jnp.asarray(_EPS))
    weights, logits, top_k_vals, top_k_indices = out
    return weights, logits, top_k_vals, top_k_indices


def kernel(ip_x, w_gating, b_gating, w_noise, b_noise, top_k):
    del top_k
    return _run(ip_x, w_gating, b_gating, w_noise, b_noise)
